# baseline (speedup 1.0000x reference)
import functools

import jax
import jax.numpy as jnp
from jax import lax
from jax.experimental import pallas as pl
from jax.experimental.pallas import tpu as pltpu
from jax.experimental.pallas import tpu_sc as plsc

_K = 8
_TN = 256
_QC = 16
_NW = 32


def _phase1(qa_ref, ka_ref, qm_ref, km_ref, fidx_ref, wts_ref, idx_ref):
    b = pl.program_id(0)
    q = qa_ref[0]
    k8s = ka_ref[0]
    qn = jnp.sum(q * q, axis=1, keepdims=True)
    kn = jnp.sum(k8s * k8s, axis=0, keepdims=True)
    qk = jnp.dot(q, k8s, preferred_element_type=jnp.float32)
    d2 = qn + kn - 2.0 * qk
    km = km_ref[0]
    d2 = jnp.where(km > 0.0, d2, jnp.inf)
    vq = qm_ref[0] > 0.0
    TN, S = d2.shape
    iota = jax.lax.broadcasted_iota(jnp.int32, (TN, S), 1)
    eps = jnp.float32(jnp.finfo(jnp.float32).eps)
    norm = jnp.zeros((TN, 1), jnp.float32)
    cols, rks = [], []
    d2m = d2
    for _ in range(_K):
        m = jnp.min(d2m, axis=1, keepdims=True)
        eq = d2m == m
        am = jnp.min(jnp.where(eq, iota, S), axis=1, keepdims=True)
        d2m = jnp.where(iota == am, jnp.inf, d2m)
        dk = jnp.where(vq, m, 0.0)
        ik = jnp.where(vq, am, 0)
        rk = 1.0 / (dk + eps)
        norm = norm + rk
        cols.append(ik)
        rks.append(rk)
    idx = jnp.concatenate(cols, axis=1)
    idx_ref[0] = idx
    fidx_ref[0] = idx + b * S
    wts_ref[0] = jnp.concatenate(rks, axis=1) / norm


def _sc_interp(fidx_hbm, wts_hbm, p2f_hbm, out_hbm,
               idx_v, w_v, rows_v, out_v, sem):
    c = lax.axis_index("c")
    s = lax.axis_index("s")
    wid = s * 2 + c
    total_q = out_hbm.shape[0]
    qw = total_q // _NW
    nchunk = qw // _QC
    base_q = wid * qw

    def chunk_body(ci, carry):
        q0 = base_q + ci * _QC
        pltpu.sync_copy(fidx_hbm.at[pl.ds(q0 * _K, _QC * _K)], idx_v)
        pltpu.sync_copy(wts_hbm.at[pl.ds(q0 * _K, _QC * _K)], w_v)
        pltpu.async_copy(p2f_hbm.at[idx_v], rows_v, sem).wait()
        for qp in range(_QC // 2):
            wv = w_v[pl.ds(qp * 16, 16)]
            for half in range(2):
                qq = qp * 2 + half
                for v in range(8):
                    sl = pl.ds(v * 16, 16)
                    a = wv[half * 8] * rows_v[qq * _K, sl]
                    for k in range(1, _K):
                        a = a + wv[half * 8 + k] * rows_v[qq * _K + k, sl]
                    out_v[qq, sl] = a
        pltpu.sync_copy(out_v, out_hbm.at[pl.ds(q0, _QC)])
        return carry

    lax.fori_loop(0, nchunk, chunk_body, 0)


def _phase2(p1_ref, interp_ref, qm_ref, w1t_ref, w2t_ref, y_ref, stats_ref):
    b = pl.program_id(0)
    i = pl.program_id(1)
    y = (jnp.dot(p1_ref[0], w1t_ref[...], preferred_element_type=jnp.float32)
         + jnp.dot(interp_ref[0], w2t_ref[...],
                   preferred_element_type=jnp.float32))
    y_ref[0] = y
    vqf = qm_ref[0]
    ym = y * vqf
    st = jnp.concatenate(
        [jnp.sum(ym, axis=0, keepdims=True),
         jnp.sum(y * ym, axis=0, keepdims=True),
         jnp.zeros((6, y.shape[1]), jnp.float32)], axis=0)
    first = jnp.logical_and(b == 0, i == 0)

    @pl.when(first)
    def _():
        stats_ref[...] = st

    @pl.when(jnp.logical_not(first))
    def _():
        stats_ref[...] = stats_ref[...] + st


def _phase3(y_ref, sb_ref, o_ref):
    o_ref[0] = y_ref[0] * sb_ref[0:1, :] + sb_ref[1:2, :]


def kernel(xyz1, xyz2, points1, points2, point_lens, embedding_lens,
           point_mask, W, gamma, beta):
    B, N, _ = xyz1.shape
    S = xyz2.shape[1]
    D1 = points1.shape[2]
    D2 = points2.shape[2]
    C = W.shape[0]
    f32 = jnp.float32

    qa = jnp.concatenate(
        [xyz1[..., :3], jnp.zeros((B, N, 5), f32)], axis=-1)
    ka = jnp.transpose(
        jnp.concatenate([xyz2, jnp.zeros((B, S, 5), f32)], axis=-1),
        (0, 2, 1))
    p1p = jnp.concatenate(
        [points1, jnp.zeros((B, N, 8 - D1), f32)], axis=-1)
    qm = point_mask.astype(f32).reshape(B, N, 1)
    km = (jnp.arange(S)[None, :] < embedding_lens[:, None]
          ).astype(f32).reshape(B, 1, S)
    w1t = jnp.concatenate(
        [W[:, :D1].T, jnp.zeros((8 - D1, C), f32)], axis=0)
    w2t = W[:, D1:].T

    fidx, wts, idx = pl.pallas_call(
        _phase1,
        grid=(B, N // _TN),
        in_specs=[
            pl.BlockSpec((1, _TN, 8), lambda b, i: (b, i, 0)),
            pl.BlockSpec((1, 8, S), lambda b, i: (b, 0, 0)),
            pl.BlockSpec((1, _TN, 1), lambda b, i: (b, i, 0)),
            pl.BlockSpec((1, 1, S), lambda b, i: (b, 0, 0)),
        ],
        out_specs=[
            pl.BlockSpec((1, _TN, _K), lambda b, i: (b, i, 0)),
            pl.BlockSpec((1, _TN, _K), lambda b, i: (b, i, 0)),
            pl.BlockSpec((1, _TN, _K), lambda b, i: (b, i, 0)),
        ],
        out_shape=[
            jax.ShapeDtypeStruct((B, N, _K), jnp.int32),
            jax.ShapeDtypeStruct((B, N, _K), f32),
            jax.ShapeDtypeStruct((B, N, _K), jnp.int32),
        ],
    )(qa, ka, qm, km)

    p2f = points2.reshape(B * S, D2)
    fidx_flat = fidx.reshape(B * N * _K)
    wts_flat = wts.reshape(B * N * _K)

    mesh = plsc.VectorSubcoreMesh(
        core_axis_name="c", subcore_axis_name="s",
        num_cores=2, num_subcores=16)
    sc_call = functools.partial(
        pl.kernel,
        out_type=jax.ShapeDtypeStruct((B * N, D2), f32),
        mesh=mesh,
        scratch_types=[
            pltpu.VMEM((_QC * _K,), jnp.int32),
            pltpu.VMEM((_QC * _K,), f32),
            pltpu.VMEM((_QC * _K, D2), f32),
            pltpu.VMEM((_QC, D2), f32),
            pltpu.SemaphoreType.DMA,
        ],
    )(_sc_interp)
    interp = sc_call(fidx_flat, wts_flat, p2f).reshape(B, N, D2)

    y, stats = pl.pallas_call(
        _phase2,
        grid=(B, N // 512),
        in_specs=[
            pl.BlockSpec((1, 512, 8), lambda b, i: (b, i, 0)),
            pl.BlockSpec((1, 512, D2), lambda b, i: (b, i, 0)),
            pl.BlockSpec((1, 512, 1), lambda b, i: (b, i, 0)),
            pl.BlockSpec((8, C), lambda b, i: (0, 0)),
            pl.BlockSpec((D2, C), lambda b, i: (0, 0)),
        ],
        out_specs=[
            pl.BlockSpec((1, 512, C), lambda b, i: (b, i, 0)),
            pl.BlockSpec((8, C), lambda b, i: (0, 0)),
        ],
        out_shape=[
            jax.ShapeDtypeStruct((B, N, C), f32),
            jax.ShapeDtypeStruct((8, C), f32),
        ],
    )(p1p, interp, qm, w1t, w2t)

    cnt = jnp.sum(point_mask.astype(f32))
    mean = stats[0] / cnt
    var = stats[1] / cnt - mean * mean
    scale = gamma / jnp.sqrt(var + 1e-5)
    bias = beta - mean * scale
    sb = jnp.stack([scale, bias], axis=0)

    out = pl.pallas_call(
        _phase3,
        grid=(B, N // 512),
        in_specs=[
            pl.BlockSpec((1, 512, C), lambda b, i: (b, i, 0)),
            pl.BlockSpec((2, C), lambda b, i: (0, 0)),
        ],
        out_specs=pl.BlockSpec((1, 512, C), lambda b, i: (b, i, 0)),
        out_shape=jax.ShapeDtypeStruct((B, N, C), f32),
    )(y, sb)
    return out, idx


# speedup vs baseline: 4.0340x; 4.0340x over previous
import jax
import jax.numpy as jnp
from jax.experimental import pallas as pl

_K = 8
_TN = 256


def _phase1(qa_ref, ka_ref, qn_ref, kn_ref, p1_ref, p2_ref, qm_ref,
            km_ref, iotaf_ref, w1t_ref, w2t_ref, y_ref, idx_ref,
            stats_ref):
    b = pl.program_id(0)
    i = pl.program_id(1)
    q = qa_ref[0]
    k8s = ka_ref[0]
    qn = qn_ref[0]
    kn = kn_ref[0]
    qk = jnp.dot(q, k8s, preferred_element_type=jnp.float32)
    d2 = qn + kn - 2.0 * qk
    km = km_ref[0]
    d2 = jnp.where(km > 0.0, d2, jnp.inf)
    vq = qm_ref[0] > 0.0
    TN, S = d2.shape
    iotaf = iotaf_ref[0]
    eps = jnp.float32(jnp.finfo(jnp.float32).eps)
    norm = jnp.zeros((TN, 1), jnp.float32)
    cols = []
    d2m = d2
    for _ in range(_K):
        m = jnp.min(d2m, axis=1, keepdims=True)
        eq = d2m == m
        amf = jnp.min(jnp.where(eq, iotaf, jnp.float32(S)), axis=1,
                      keepdims=True)
        d2m = jnp.where(iotaf == amf, jnp.inf, d2m)
        dk = jnp.where(vq, m, 0.0)
        ikf = jnp.where(vq, amf, 0.0)
        rk = 1.0 / (dk + eps)
        norm = norm + rk
        cols.append(ikf)
    idx_ref[0] = jnp.concatenate(cols, axis=1)
    sel = jnp.logical_and(d2m == jnp.inf, d2 != jnp.inf)
    rmat = jnp.where(sel, 1.0 / (d2 + eps), 0.0)
    wmat = jnp.where(vq, rmat / norm,
                     jnp.where(iotaf == 0.0, 1.0, 0.0))
    interp = jnp.dot(wmat, p2_ref[0], preferred_element_type=jnp.float32)
    y = (jnp.dot(p1_ref[0], w1t_ref[...], preferred_element_type=jnp.float32)
         + jnp.dot(interp, w2t_ref[...], preferred_element_type=jnp.float32))
    y_ref[0] = y
    vqf = qm_ref[0]
    ym = y * vqf
    s = jnp.sum(ym, axis=0, keepdims=True)
    ss = jnp.sum(y * ym, axis=0, keepdims=True)
    st = jnp.concatenate(
        [s, ss, jnp.zeros((6, s.shape[1]), jnp.float32)], axis=0)

    first = jnp.logical_and(b == 0, i == 0)

    @pl.when(first)
    def _():
        stats_ref[...] = st

    @pl.when(jnp.logical_not(first))
    def _():
        stats_ref[...] = stats_ref[...] + st


def _phase2(y_ref, sb_ref, o_ref):
    o_ref[0] = y_ref[0] * sb_ref[0:1, :] + sb_ref[1:2, :]


def kernel(xyz1, xyz2, points1, points2, point_lens, embedding_lens,
           point_mask, W, gamma, beta):
    B, N, _ = xyz1.shape
    S = xyz2.shape[1]
    D1 = points1.shape[2]
    D2 = points2.shape[2]
    C = W.shape[0]
    f32 = jnp.float32

    qa = jnp.concatenate(
        [xyz1[..., :3], jnp.zeros((B, N, 5), f32)], axis=-1)
    ka = jnp.transpose(
        jnp.concatenate([xyz2, jnp.zeros((B, S, 5), f32)], axis=-1),
        (0, 2, 1))
    p1p = jnp.concatenate(
        [points1, jnp.zeros((B, N, 8 - D1), f32)], axis=-1)
    qm = point_mask.astype(f32).reshape(B, N, 1)
    km = (jnp.arange(S)[None, :] < embedding_lens[:, None]
          ).astype(f32).reshape(B, 1, S)
    q3 = xyz1[..., :3]
    qn_in = (q3 * q3).sum(-1).reshape(B, N, 1)
    kn_in = (xyz2 * xyz2).sum(-1).reshape(B, 1, S)
    iotaf_in = jnp.arange(S, dtype=f32).reshape(1, 1, S)
    w1t = jnp.concatenate(
        [W[:, :D1].T, jnp.zeros((8 - D1, C), f32)], axis=0)
    w2t = W[:, D1:].T

    y, idx, stats = pl.pallas_call(
        _phase1,
        grid=(B, N // _TN),
        in_specs=[
            pl.BlockSpec((1, _TN, 8), lambda b, i: (b, i, 0)),
            pl.BlockSpec((1, 8, S), lambda b, i: (b, 0, 0)),
            pl.BlockSpec((1, _TN, 1), lambda b, i: (b, i, 0)),
            pl.BlockSpec((1, 1, S), lambda b, i: (b, 0, 0)),
            pl.BlockSpec((1, _TN, 8), lambda b, i: (b, i, 0)),
            pl.BlockSpec((1, S, D2), lambda b, i: (b, 0, 0)),
            pl.BlockSpec((1, _TN, 1), lambda b, i: (b, i, 0)),
            pl.BlockSpec((1, 1, S), lambda b, i: (b, 0, 0)),
            pl.BlockSpec((1, 1, S), lambda b, i: (0, 0, 0)),
            pl.BlockSpec((8, C), lambda b, i: (0, 0)),
            pl.BlockSpec((D2, C), lambda b, i: (0, 0)),
        ],
        out_specs=[
            pl.BlockSpec((1, _TN, C), lambda b, i: (b, i, 0)),
            pl.BlockSpec((1, _TN, _K), lambda b, i: (b, i, 0)),
            pl.BlockSpec((8, C), lambda b, i: (0, 0)),
        ],
        out_shape=[
            jax.ShapeDtypeStruct((B, N, C), f32),
            jax.ShapeDtypeStruct((B, N, _K), f32),
            jax.ShapeDtypeStruct((8, C), f32),
        ],
    )(qa, ka, qn_in, kn_in, p1p, points2, qm, km, iotaf_in, w1t, w2t)
    idx = idx.astype(jnp.int32)

    cnt = jnp.sum(point_mask.astype(f32))
    mean = stats[0] / cnt
    var = stats[1] / cnt - mean * mean
    scale = gamma / jnp.sqrt(var + 1e-5)
    bias = beta - mean * scale
    sb = jnp.stack([scale, bias], axis=0)

    tn2 = 512
    out = pl.pallas_call(
        _phase2,
        grid=(B, N // tn2),
        in_specs=[
            pl.BlockSpec((1, tn2, C), lambda b, i: (b, i, 0)),
            pl.BlockSpec((2, C), lambda b, i: (0, 0)),
        ],
        out_specs=pl.BlockSpec((1, tn2, C), lambda b, i: (b, i, 0)),
        out_shape=jax.ShapeDtypeStruct((B, N, C), f32),
    )(y, sb)
    return out, idx


# speedup vs baseline: 4.1199x; 1.0213x over previous
import jax
import jax.numpy as jnp
from jax.experimental import pallas as pl

_K = 8
_TN = 512


def _phase1(qa_ref, ka_ref, qn_ref, kn_ref, p1_ref, p2_ref, qm_ref,
            km_ref, iotaf_ref, w1t_ref, w2t_ref, y_ref, idx_ref,
            stats_ref):
    b = pl.program_id(0)
    i = pl.program_id(1)
    q = qa_ref[0]
    k8s = ka_ref[0]
    qn = qn_ref[0]
    kn = kn_ref[0]
    qk = jnp.dot(q, k8s, preferred_element_type=jnp.float32)
    d2 = qn + kn - 2.0 * qk
    km = km_ref[0]
    d2 = jnp.where(km > 0.0, d2, jnp.inf)
    vq = qm_ref[0] > 0.0
    TN, S = d2.shape
    iotaf = iotaf_ref[0]
    eps = jnp.float32(jnp.finfo(jnp.float32).eps)
    norm = jnp.zeros((TN, 1), jnp.float32)
    cols = []
    d2m = d2
    for _ in range(_K):
        m = jnp.min(d2m, axis=1, keepdims=True)
        eq = d2m == m
        amf = jnp.min(jnp.where(eq, iotaf, jnp.float32(S)), axis=1,
                      keepdims=True)
        d2m = jnp.where(iotaf == amf, jnp.inf, d2m)
        dk = jnp.where(vq, m, 0.0)
        ikf = jnp.where(vq, amf, 0.0)
        rk = 1.0 / (dk + eps)
        norm = norm + rk
        cols.append(ikf)
    idx_ref[0] = jnp.concatenate(cols, axis=1)
    sel = jnp.logical_and(d2m == jnp.inf, d2 != jnp.inf)
    rmat = jnp.where(sel, 1.0 / (d2 + eps), 0.0)
    wmat = jnp.where(vq, rmat / norm,
                     jnp.where(iotaf == 0.0, 1.0, 0.0))
    interp = jnp.dot(wmat, p2_ref[0], preferred_element_type=jnp.float32)
    y = (jnp.dot(p1_ref[0], w1t_ref[...], preferred_element_type=jnp.float32)
         + jnp.dot(interp, w2t_ref[...], preferred_element_type=jnp.float32))
    y_ref[0] = y
    vqf = qm_ref[0]
    ym = y * vqf
    s = jnp.sum(ym, axis=0, keepdims=True)
    ss = jnp.sum(y * ym, axis=0, keepdims=True)
    st = jnp.concatenate(
        [s, ss, jnp.zeros((6, s.shape[1]), jnp.float32)], axis=0)

    first = jnp.logical_and(b == 0, i == 0)

    @pl.when(first)
    def _():
        stats_ref[...] = st

    @pl.when(jnp.logical_not(first))
    def _():
        stats_ref[...] = stats_ref[...] + st


def _phase2(y_ref, sb_ref, o_ref):
    o_ref[0] = y_ref[0] * sb_ref[0:1, :] + sb_ref[1:2, :]


def kernel(xyz1, xyz2, points1, points2, point_lens, embedding_lens,
           point_mask, W, gamma, beta):
    B, N, _ = xyz1.shape
    S = xyz2.shape[1]
    D1 = points1.shape[2]
    D2 = points2.shape[2]
    C = W.shape[0]
    f32 = jnp.float32

    qa = jnp.concatenate(
        [xyz1[..., :3], jnp.zeros((B, N, 5), f32)], axis=-1)
    ka = jnp.transpose(
        jnp.concatenate([xyz2, jnp.zeros((B, S, 5), f32)], axis=-1),
        (0, 2, 1))
    p1p = jnp.concatenate(
        [points1, jnp.zeros((B, N, 8 - D1), f32)], axis=-1)
    qm = point_mask.astype(f32).reshape(B, N, 1)
    km = (jnp.arange(S)[None, :] < embedding_lens[:, None]
          ).astype(f32).reshape(B, 1, S)
    q3 = xyz1[..., :3]
    qn_in = (q3 * q3).sum(-1).reshape(B, N, 1)
    kn_in = (xyz2 * xyz2).sum(-1).reshape(B, 1, S)
    iotaf_in = jnp.arange(S, dtype=f32).reshape(1, 1, S)
    w1t = jnp.concatenate(
        [W[:, :D1].T, jnp.zeros((8 - D1, C), f32)], axis=0)
    w2t = W[:, D1:].T

    y, idx, stats = pl.pallas_call(
        _phase1,
        grid=(B, N // _TN),
        in_specs=[
            pl.BlockSpec((1, _TN, 8), lambda b, i: (b, i, 0)),
            pl.BlockSpec((1, 8, S), lambda b, i: (b, 0, 0)),
            pl.BlockSpec((1, _TN, 1), lambda b, i: (b, i, 0)),
            pl.BlockSpec((1, 1, S), lambda b, i: (b, 0, 0)),
            pl.BlockSpec((1, _TN, 8), lambda b, i: (b, i, 0)),
            pl.BlockSpec((1, S, D2), lambda b, i: (b, 0, 0)),
            pl.BlockSpec((1, _TN, 1), lambda b, i: (b, i, 0)),
            pl.BlockSpec((1, 1, S), lambda b, i: (b, 0, 0)),
            pl.BlockSpec((1, 1, S), lambda b, i: (0, 0, 0)),
            pl.BlockSpec((8, C), lambda b, i: (0, 0)),
            pl.BlockSpec((D2, C), lambda b, i: (0, 0)),
        ],
        out_specs=[
            pl.BlockSpec((1, _TN, C), lambda b, i: (b, i, 0)),
            pl.BlockSpec((1, _TN, _K), lambda b, i: (b, i, 0)),
            pl.BlockSpec((8, C), lambda b, i: (0, 0)),
        ],
        out_shape=[
            jax.ShapeDtypeStruct((B, N, C), f32),
            jax.ShapeDtypeStruct((B, N, _K), f32),
            jax.ShapeDtypeStruct((8, C), f32),
        ],
    )(qa, ka, qn_in, kn_in, p1p, points2, qm, km, iotaf_in, w1t, w2t)
    idx = idx.astype(jnp.int32)

    cnt = jnp.sum(point_mask.astype(f32))
    mean = stats[0] / cnt
    var = stats[1] / cnt - mean * mean
    scale = gamma / jnp.sqrt(var + 1e-5)
    bias = beta - mean * scale
    sb = jnp.stack([scale, bias], axis=0)

    tn2 = 512
    out = pl.pallas_call(
        _phase2,
        grid=(B, N // tn2),
        in_specs=[
            pl.BlockSpec((1, tn2, C), lambda b, i: (b, i, 0)),
            pl.BlockSpec((2, C), lambda b, i: (0, 0)),
        ],
        out_specs=pl.BlockSpec((1, tn2, C), lambda b, i: (b, i, 0)),
        out_shape=jax.ShapeDtypeStruct((B, N, C), f32),
    )(y, sb)
    return out, idx


# speedup vs baseline: 4.4494x; 1.0800x over previous
import jax
import jax.numpy as jnp
from jax.experimental import pallas as pl

_K = 8
_TN = 1024


def _phase1(qa_ref, ka_ref, qn_ref, kn_ref, p1_ref, p2_ref, qm_ref,
            km_ref, iotaf_ref, w1t_ref, w2t_ref, y_ref, idx_ref,
            stats_ref):
    b = pl.program_id(0)
    i = pl.program_id(1)
    q = qa_ref[0]
    k8s = ka_ref[0]
    qn = qn_ref[0]
    kn = kn_ref[0]
    qk = jnp.dot(q, k8s, preferred_element_type=jnp.float32)
    d2 = qn + kn - 2.0 * qk
    km = km_ref[0]
    d2 = jnp.where(km > 0.0, d2, jnp.inf)
    vq = qm_ref[0] > 0.0
    TN, S = d2.shape
    iotaf = iotaf_ref[0]
    eps = jnp.float32(jnp.finfo(jnp.float32).eps)
    norm = jnp.zeros((TN, 1), jnp.float32)
    cols = []
    d2m = d2
    for _ in range(_K):
        m = jnp.min(d2m, axis=1, keepdims=True)
        eq = d2m == m
        amf = jnp.min(jnp.where(eq, iotaf, jnp.float32(S)), axis=1,
                      keepdims=True)
        d2m = jnp.where(iotaf == amf, jnp.inf, d2m)
        dk = jnp.where(vq, m, 0.0)
        ikf = jnp.where(vq, amf, 0.0)
        rk = 1.0 / (dk + eps)
        norm = norm + rk
        cols.append(ikf)
    idx_ref[0] = jnp.concatenate(cols, axis=1)
    sel = jnp.logical_and(d2m == jnp.inf, d2 != jnp.inf)
    rmat = jnp.where(sel, 1.0 / (d2 + eps), 0.0)
    wmat = jnp.where(vq, rmat / norm,
                     jnp.where(iotaf == 0.0, 1.0, 0.0))
    interp = jnp.dot(wmat, p2_ref[0], preferred_element_type=jnp.float32)
    y = (jnp.dot(p1_ref[0], w1t_ref[...], preferred_element_type=jnp.float32)
         + jnp.dot(interp, w2t_ref[...], preferred_element_type=jnp.float32))
    y_ref[0] = y
    vqf = qm_ref[0]
    ym = y * vqf
    s = jnp.sum(ym, axis=0, keepdims=True)
    ss = jnp.sum(y * ym, axis=0, keepdims=True)
    st = jnp.concatenate(
        [s, ss, jnp.zeros((6, s.shape[1]), jnp.float32)], axis=0)

    first = jnp.logical_and(b == 0, i == 0)

    @pl.when(first)
    def _():
        stats_ref[...] = st

    @pl.when(jnp.logical_not(first))
    def _():
        stats_ref[...] = stats_ref[...] + st


def _phase2(y_ref, sb_ref, o_ref):
    o_ref[0] = y_ref[0] * sb_ref[0:1, :] + sb_ref[1:2, :]


def kernel(xyz1, xyz2, points1, points2, point_lens, embedding_lens,
           point_mask, W, gamma, beta):
    B, N, _ = xyz1.shape
    S = xyz2.shape[1]
    D1 = points1.shape[2]
    D2 = points2.shape[2]
    C = W.shape[0]
    f32 = jnp.float32

    qa = jnp.concatenate(
        [xyz1[..., :3], jnp.zeros((B, N, 5), f32)], axis=-1)
    ka = jnp.transpose(
        jnp.concatenate([xyz2, jnp.zeros((B, S, 5), f32)], axis=-1),
        (0, 2, 1))
    p1p = jnp.concatenate(
        [points1, jnp.zeros((B, N, 8 - D1), f32)], axis=-1)
    qm = point_mask.astype(f32).reshape(B, N, 1)
    km = (jnp.arange(S)[None, :] < embedding_lens[:, None]
          ).astype(f32).reshape(B, 1, S)
    q3 = xyz1[..., :3]
    qn_in = (q3 * q3).sum(-1).reshape(B, N, 1)
    kn_in = (xyz2 * xyz2).sum(-1).reshape(B, 1, S)
    iotaf_in = jnp.arange(S, dtype=f32).reshape(1, 1, S)
    w1t = jnp.concatenate(
        [W[:, :D1].T, jnp.zeros((8 - D1, C), f32)], axis=0)
    w2t = W[:, D1:].T

    y, idx, stats = pl.pallas_call(
        _phase1,
        grid=(B, N // _TN),
        in_specs=[
            pl.BlockSpec((1, _TN, 8), lambda b, i: (b, i, 0)),
            pl.BlockSpec((1, 8, S), lambda b, i: (b, 0, 0)),
            pl.BlockSpec((1, _TN, 1), lambda b, i: (b, i, 0)),
            pl.BlockSpec((1, 1, S), lambda b, i: (b, 0, 0)),
            pl.BlockSpec((1, _TN, 8), lambda b, i: (b, i, 0)),
            pl.BlockSpec((1, S, D2), lambda b, i: (b, 0, 0)),
            pl.BlockSpec((1, _TN, 1), lambda b, i: (b, i, 0)),
            pl.BlockSpec((1, 1, S), lambda b, i: (b, 0, 0)),
            pl.BlockSpec((1, 1, S), lambda b, i: (0, 0, 0)),
            pl.BlockSpec((8, C), lambda b, i: (0, 0)),
            pl.BlockSpec((D2, C), lambda b, i: (0, 0)),
        ],
        out_specs=[
            pl.BlockSpec((1, _TN, C), lambda b, i: (b, i, 0)),
            pl.BlockSpec((1, _TN, _K), lambda b, i: (b, i, 0)),
            pl.BlockSpec((8, C), lambda b, i: (0, 0)),
        ],
        out_shape=[
            jax.ShapeDtypeStruct((B, N, C), f32),
            jax.ShapeDtypeStruct((B, N, _K), f32),
            jax.ShapeDtypeStruct((8, C), f32),
        ],
    )(qa, ka, qn_in, kn_in, p1p, points2, qm, km, iotaf_in, w1t, w2t)
    idx = idx.astype(jnp.int32)

    cnt = jnp.sum(point_mask.astype(f32))
    mean = stats[0] / cnt
    var = stats[1] / cnt - mean * mean
    scale = gamma / jnp.sqrt(var + 1e-5)
    bias = beta - mean * scale
    sb = jnp.stack([scale, bias], axis=0)

    tn2 = 512
    out = pl.pallas_call(
        _phase2,
        grid=(B, N // tn2),
        in_specs=[
            pl.BlockSpec((1, tn2, C), lambda b, i: (b, i, 0)),
            pl.BlockSpec((2, C), lambda b, i: (0, 0)),
        ],
        out_specs=pl.BlockSpec((1, tn2, C), lambda b, i: (b, i, 0)),
        out_shape=jax.ShapeDtypeStruct((B, N, C), f32),
    )(y, sb)
    return out, idx
